# baseline (speedup 1.0000x reference)
import functools

import jax
import jax.numpy as jnp
from jax.experimental import pallas as pl
from jax.experimental.pallas import tpu as pltpu


def _patches(xf, wdim, hw):
    cin = xf.shape[0]
    zpad = jnp.zeros((cin, wdim + 1), xf.dtype)
    xfp = jnp.concatenate([zpad, xf, zpad], axis=1)
    col = jax.lax.broadcasted_iota(jnp.int32, (1, hw), 1) % wdim
    not_left = col != 0
    not_right = col != (wdim - 1)
    zero = jnp.zeros((), xf.dtype)
    slices = []
    for dy in range(3):
        for dx in range(3):
            start = dy * wdim + dx
            s = jax.lax.slice(xfp, (0, start), (cin, start + hw))
            if dx == 0:
                s = jnp.where(not_left, s, zero)
            elif dx == 2:
                s = jnp.where(not_right, s, zero)
            slices.append(s)
    return jnp.concatenate(slices, axis=0)


def _conv_block(x_ref, w_ref, i, wdim, hw):
    cin = x_ref.shape[1]
    xf = x_ref[i].astype(jnp.bfloat16).reshape(cin, hw)
    p = _patches(xf, wdim, hw)
    return jax.lax.dot_general(w_ref[...], p, (((1,), (0,)), ((), ())),
                               preferred_element_type=jnp.float32)


def _stats_kernel(x_ref, w_ref, s_ref, q_ref, *, wdim, hw, nb):
    cout = w_ref.shape[0]
    s = jnp.zeros((cout, 1), jnp.float32)
    q = jnp.zeros((cout, 1), jnp.float32)
    for i in range(nb):
        y = _conv_block(x_ref, w_ref, i, wdim, hw)
        s = s + jnp.sum(y, axis=1, keepdims=True)
        q = q + jnp.sum(y * y, axis=1, keepdims=True)
    s_ref[0] = s
    q_ref[0] = q


def _conv_bn_kernel(x_ref, w_ref, s_ref, q_ref, o_ref, *, wdim, hw, m_total,
                    eps, neg_slope, nb):
    hdim = hw // wdim
    cout = o_ref.shape[1]
    inv_m = jnp.float32(1.0 / m_total)
    mean = jnp.sum(s_ref[...], axis=0) * inv_m
    msq = jnp.sum(q_ref[...], axis=0) * inv_m
    var = msq - mean * mean
    scale = jax.lax.rsqrt(var + eps)
    shift = -mean * scale
    for i in range(nb):
        y = _conv_block(x_ref, w_ref, i, wdim, hw)
        yh = y * scale + shift
        yh = jnp.where(yh >= 0, yh, neg_slope * yh)
        o_ref[i] = yh.reshape(cout, hdim, wdim)


def kernel(x, w, b):
    del b
    n, cin, h, wdim = x.shape
    cout = w.shape[0]
    hw = h * wdim
    k = 9 * cin
    m_total = n * hw
    nb = 2 if n % 2 == 0 else 1
    nblk = n // nb

    wr = w.transpose(0, 2, 3, 1).reshape(cout, k).astype(jnp.bfloat16)

    cp = pltpu.CompilerParams(
        dimension_semantics=("parallel",),
        vmem_limit_bytes=56 * 1024 * 1024,
    )

    psum, pq = pl.pallas_call(
        functools.partial(_stats_kernel, wdim=wdim, hw=hw, nb=nb),
        out_shape=(
            jax.ShapeDtypeStruct((nblk, cout, 1), jnp.float32),
            jax.ShapeDtypeStruct((nblk, cout, 1), jnp.float32),
        ),
        grid=(nblk,),
        in_specs=[
            pl.BlockSpec((nb, cin, h, wdim), lambda i: (i, 0, 0, 0)),
            pl.BlockSpec((cout, k), lambda i: (0, 0)),
        ],
        out_specs=(
            pl.BlockSpec((1, cout, 1), lambda i: (i, 0, 0)),
            pl.BlockSpec((1, cout, 1), lambda i: (i, 0, 0)),
        ),
        compiler_params=cp,
    )(x, wr)

    out = pl.pallas_call(
        functools.partial(_conv_bn_kernel, wdim=wdim, hw=hw, m_total=m_total,
                          eps=1e-5, neg_slope=0.01, nb=nb),
        out_shape=jax.ShapeDtypeStruct((n, cout, h, wdim), jnp.float32),
        grid=(nblk,),
        in_specs=[
            pl.BlockSpec((nb, cin, h, wdim), lambda i: (i, 0, 0, 0)),
            pl.BlockSpec((cout, k), lambda i: (0, 0)),
            pl.BlockSpec((nblk, cout, 1), lambda i: (0, 0, 0)),
            pl.BlockSpec((nblk, cout, 1), lambda i: (0, 0, 0)),
        ],
        out_specs=pl.BlockSpec((nb, cout, h, wdim), lambda i: (i, 0, 0, 0)),
        compiler_params=cp,
    )(x, wr, psum, pq)

    return out


# speedup vs baseline: 1.5183x; 1.5183x over previous
import functools

import jax
import jax.numpy as jnp
from jax.experimental import pallas as pl
from jax.experimental.pallas import tpu as pltpu


def _patches(xf, wdim, hw):
    cin = xf.shape[0]
    zpad = jnp.zeros((cin, wdim + 1), xf.dtype)
    xfp = jnp.concatenate([zpad, xf, zpad], axis=1)
    col = jax.lax.broadcasted_iota(jnp.int32, (1, hw), 1) % wdim
    not_left = col != 0
    not_right = col != (wdim - 1)
    zero = jnp.zeros((), xf.dtype)
    slices = []
    for dy in range(3):
        for dx in range(3):
            start = dy * wdim + dx
            s = jax.lax.slice(xfp, (0, start), (cin, start + hw))
            if dx == 0:
                s = jnp.where(not_left, s, zero)
            elif dx == 2:
                s = jnp.where(not_right, s, zero)
            slices.append(s)
    return jnp.concatenate(slices, axis=0)


def _conv_stats_kernel(x_ref, w_ref, ones_ref, y_ref, s_ref, q_ref, *,
                       wdim, hw, nb):
    cin = x_ref.shape[1]
    cout = w_ref.shape[0]
    s = jnp.zeros((cout, 1), jnp.float32)
    q = jnp.zeros((cout, 1), jnp.float32)
    for i in range(nb):
        xf = x_ref[i].astype(jnp.bfloat16).reshape(cin, hw)
        p = _patches(xf, wdim, hw)
        y = jax.lax.dot_general(w_ref[...], p, (((1,), (0,)), ((), ())),
                                preferred_element_type=jnp.float32)
        yb = y.astype(jnp.bfloat16)
        y2b = (y * y).astype(jnp.bfloat16)
        y_ref[i] = yb
        rs = jax.lax.dot_general(yb, ones_ref[...], (((1,), (0,)), ((), ())),
                                 preferred_element_type=jnp.float32)
        rq = jax.lax.dot_general(y2b, ones_ref[...], (((1,), (0,)), ((), ())),
                                 preferred_element_type=jnp.float32)
        s = s + rs[:, 0:1]
        q = q + rq[:, 0:1]
    s_ref[0] = s
    q_ref[0] = q


def _bn_lrelu_kernel(y_ref, s_ref, q_ref, o_ref, *, m_total, eps, neg_slope):
    inv_m = jnp.float32(1.0 / m_total)
    mean = jnp.sum(s_ref[...], axis=0) * inv_m
    msq = jnp.sum(q_ref[...], axis=0) * inv_m
    var = msq - mean * mean
    scale = jax.lax.rsqrt(var + eps)
    shift = -mean * scale
    yh = y_ref[...].astype(jnp.float32) * scale + shift
    o_ref[...] = jnp.where(yh >= 0, yh, neg_slope * yh)


def kernel(x, w, b):
    del b
    n, cin, h, wdim = x.shape
    cout = w.shape[0]
    hw = h * wdim
    k = 9 * cin
    m_total = n * hw
    nb = 4 if n % 4 == 0 else 1
    nblk = n // nb

    wr = w.transpose(0, 2, 3, 1).reshape(cout, k).astype(jnp.bfloat16)
    ones_r = jnp.ones((hw, 128), jnp.bfloat16)

    cp = pltpu.CompilerParams(
        dimension_semantics=("parallel",),
        vmem_limit_bytes=56 * 1024 * 1024,
    )

    yb, psum, pq = pl.pallas_call(
        functools.partial(_conv_stats_kernel, wdim=wdim, hw=hw, nb=nb),
        out_shape=(
            jax.ShapeDtypeStruct((n, cout, hw), jnp.bfloat16),
            jax.ShapeDtypeStruct((nblk, cout, 1), jnp.float32),
            jax.ShapeDtypeStruct((nblk, cout, 1), jnp.float32),
        ),
        grid=(nblk,),
        in_specs=[
            pl.BlockSpec((nb, cin, h, wdim), lambda i: (i, 0, 0, 0)),
            pl.BlockSpec((cout, k), lambda i: (0, 0)),
            pl.BlockSpec((hw, 128), lambda i: (0, 0)),
        ],
        out_specs=(
            pl.BlockSpec((nb, cout, hw), lambda i: (i, 0, 0)),
            pl.BlockSpec((1, cout, 1), lambda i: (i, 0, 0)),
            pl.BlockSpec((1, cout, 1), lambda i: (i, 0, 0)),
        ),
        compiler_params=cp,
    )(x, wr, ones_r)

    out = pl.pallas_call(
        functools.partial(_bn_lrelu_kernel, m_total=m_total, eps=1e-5,
                          neg_slope=0.01),
        out_shape=jax.ShapeDtypeStruct((n, cout, hw), jnp.float32),
        grid=(nblk,),
        in_specs=[
            pl.BlockSpec((nb, cout, hw), lambda i: (i, 0, 0)),
            pl.BlockSpec((nblk, cout, 1), lambda i: (0, 0, 0)),
            pl.BlockSpec((nblk, cout, 1), lambda i: (0, 0, 0)),
        ],
        out_specs=pl.BlockSpec((nb, cout, hw), lambda i: (i, 0, 0)),
        compiler_params=cp,
    )(yb, psum, pq)

    return out.reshape(n, cout, h, wdim)


# speedup vs baseline: 1.8185x; 1.1978x over previous
import functools

import jax
import jax.numpy as jnp
from jax.experimental import pallas as pl
from jax.experimental.pallas import tpu as pltpu


def _patches(xf, wdim, hw):
    cin = xf.shape[0]
    hwp = hw + 2 * wdim + 2
    zpad = jnp.zeros((cin, wdim + 1), xf.dtype)
    xfp = jnp.concatenate([zpad, xf, zpad], axis=1)
    colp = jax.lax.broadcasted_iota(jnp.int32, (1, hwp), 1) % wdim
    zero = jnp.zeros((), xf.dtype)
    xm0 = jnp.where(colp != 0, xfp, zero)
    xm2 = jnp.where(colp != 1, xfp, zero)
    srcs = (xm0, xfp, xm2)
    slices = []
    for dy in range(3):
        for dx in range(3):
            start = dy * wdim + dx
            slices.append(
                jax.lax.slice(srcs[dx], (0, start), (cin, start + hw)))
    return jnp.concatenate(slices, axis=0)


def _conv_stats_kernel(x_ref, w_ref, ones_ref, y_ref, s_ref, q_ref, *,
                       wdim, hw, nb):
    cout = w_ref.shape[0]
    s = jnp.zeros((cout, 1), jnp.float32)
    q = jnp.zeros((cout, 1), jnp.float32)
    for i in range(nb):
        xf = x_ref[i].astype(jnp.bfloat16)
        p = _patches(xf, wdim, hw)
        y = jax.lax.dot_general(w_ref[...], p, (((1,), (0,)), ((), ())),
                                preferred_element_type=jnp.float32)
        yb = y.astype(jnp.bfloat16)
        y2b = (y * y).astype(jnp.bfloat16)
        y_ref[i] = yb
        rs = jax.lax.dot_general(yb, ones_ref[...], (((1,), (0,)), ((), ())),
                                 preferred_element_type=jnp.float32)
        rq = jax.lax.dot_general(y2b, ones_ref[...], (((1,), (0,)), ((), ())),
                                 preferred_element_type=jnp.float32)
        s = s + rs[:, 0:1]
        q = q + rq[:, 0:1]
    s_ref[0] = s
    q_ref[0] = q


def _bn_lrelu_kernel(y_ref, s_ref, q_ref, o_ref, *, m_total, eps, neg_slope):
    inv_m = jnp.float32(1.0 / m_total)
    mean = jnp.sum(s_ref[...], axis=0) * inv_m
    msq = jnp.sum(q_ref[...], axis=0) * inv_m
    var = msq - mean * mean
    scale = jax.lax.rsqrt(var + eps)
    shift = -mean * scale
    yh = y_ref[...].astype(jnp.float32) * scale + shift
    o_ref[...] = jnp.where(yh >= 0, yh, neg_slope * yh)


def kernel(x, w, b):
    del b
    n, cin, h, wdim = x.shape
    cout = w.shape[0]
    hw = h * wdim
    k = 9 * cin
    m_total = n * hw
    nb = 4 if n % 4 == 0 else 1
    nblk = n // nb

    wr = w.transpose(0, 2, 3, 1).reshape(cout, k).astype(jnp.bfloat16)
    ones_r = jnp.ones((hw, 128), jnp.bfloat16)

    cp = pltpu.CompilerParams(
        dimension_semantics=("parallel",),
        vmem_limit_bytes=56 * 1024 * 1024,
    )

    yb, psum, pq = pl.pallas_call(
        functools.partial(_conv_stats_kernel, wdim=wdim, hw=hw, nb=nb),
        out_shape=(
            jax.ShapeDtypeStruct((n, cout, hw), jnp.bfloat16),
            jax.ShapeDtypeStruct((nblk, cout, 1), jnp.float32),
            jax.ShapeDtypeStruct((nblk, cout, 1), jnp.float32),
        ),
        grid=(nblk,),
        in_specs=[
            pl.BlockSpec((nb, cin, hw), lambda i: (i, 0, 0)),
            pl.BlockSpec((cout, k), lambda i: (0, 0)),
            pl.BlockSpec((hw, 128), lambda i: (0, 0)),
        ],
        out_specs=(
            pl.BlockSpec((nb, cout, hw), lambda i: (i, 0, 0)),
            pl.BlockSpec((1, cout, 1), lambda i: (i, 0, 0)),
            pl.BlockSpec((1, cout, 1), lambda i: (i, 0, 0)),
        ),
        compiler_params=cp,
    )(x.reshape(n, cin, hw), wr, ones_r)

    out = pl.pallas_call(
        functools.partial(_bn_lrelu_kernel, m_total=m_total, eps=1e-5,
                          neg_slope=0.01),
        out_shape=jax.ShapeDtypeStruct((n, cout, hw), jnp.float32),
        grid=(nblk,),
        in_specs=[
            pl.BlockSpec((nb, cout, hw), lambda i: (i, 0, 0)),
            pl.BlockSpec((nblk, cout, 1), lambda i: (0, 0, 0)),
            pl.BlockSpec((nblk, cout, 1), lambda i: (0, 0, 0)),
        ],
        out_specs=pl.BlockSpec((nb, cout, hw), lambda i: (i, 0, 0)),
        compiler_params=cp,
    )(yb, psum, pq)

    return out.reshape(n, cout, h, wdim)
